# baseline (speedup 1.0000x reference)
import functools
import math

import jax
import jax.numpy as jnp
from jax import lax
from jax.experimental import pallas as pl
from jax.experimental.pallas import tpu as pltpu
from jax.experimental.pallas import tpu_sc as plsc

E = 8
D = 1024
DFF = 4096
T = 4096

TB = 512
TBLK = 512
NB = T // TBLK + E
TP = NB * TBLK
F = 1024
NF = DFF // F

NC = 2
NS = 16
NW = NC * NS
TPW = T // NW
CH = 32
NCHUNK = TPW // CH


def _gate_kernel(x_ref, wgr_ref, wg_ref, scores_ref, eidx_ref, rank_ref,
                 counts_ref):
    i = pl.program_id(0)
    xb = x_ref[...]
    red = jnp.dot(xb, wgr_ref[...].T, preferred_element_type=jnp.float32)
    wg = wg_ref[...]
    wgu = wg * lax.rsqrt(jnp.sum(wg * wg, axis=1, keepdims=True))
    logits = jnp.dot(red, wgu.T, preferred_element_type=jnp.float32)
    m = jnp.max(logits, axis=1, keepdims=True)
    ssum = jnp.sum(jnp.exp(logits - m), axis=1, keepdims=True)
    scores_ref[...] = jnp.broadcast_to(1.0 / ssum, (TB, 128))
    it8 = lax.broadcasted_iota(jnp.int32, (TB, E), 1)
    eidx = jnp.min(jnp.where(logits == m, it8, E), axis=1, keepdims=True)
    eidx_ref[...] = eidx
    oh = (it8 == eidx).astype(jnp.float32)
    tri = (lax.broadcasted_iota(jnp.int32, (TB, TB), 0)
           >= lax.broadcasted_iota(jnp.int32, (TB, TB), 1)).astype(jnp.float32)
    cs = jnp.dot(tri, oh, preferred_element_type=jnp.float32)

    @pl.when(i == 0)
    def _():
        counts_ref[...] = jnp.zeros_like(counts_ref)

    prev = counts_ref[...]
    rank_ref[...] = jnp.sum((cs + prev) * oh, axis=1,
                            keepdims=True).astype(jnp.int32)
    counts_ref[...] = prev + jnp.sum(oh, axis=0, keepdims=True)


def _route_kernel(eidx_ref, rank_ref, counts_ref, pos_ref, be_ref):
    counts = counts_ref[...]
    padded = jnp.floor((counts + (TBLK - 1)) / TBLK) * TBLK
    triu = (lax.broadcasted_iota(jnp.int32, (E, E), 0)
            <= lax.broadcasted_iota(jnp.int32, (E, E), 1)).astype(jnp.float32)
    ends = jnp.dot(padded, triu, preferred_element_type=jnp.float32)
    starts = ends - padded
    eidx = eidx_ref[...]
    oh = (lax.broadcasted_iota(jnp.int32, (T, E), 1) == eidx).astype(jnp.float32)
    start_t = jnp.sum(oh * starts, axis=1, keepdims=True)
    pos_ref[...] = (start_t + rank_ref[...].astype(jnp.float32)
                    - 1.0).astype(jnp.int32)
    ib = (lax.broadcasted_iota(jnp.int32, (NB, E), 0) * TBLK).astype(jnp.float32)
    be_ref[...] = jnp.sum((ib >= ends).astype(jnp.int32), axis=1,
                          keepdims=True)


def _ffn_kernel(be_ref, x_ref, w1_ref, b1_ref, w2_ref, b2_ref, s_ref, out_ref):
    i = pl.program_id(0)
    j = pl.program_id(1)

    @pl.when(j == 0)
    def _():
        out_ref[...] = jnp.zeros_like(out_ref)

    @pl.when(be_ref[i] < E)
    def _():
        h = jnp.dot(x_ref[...], w1_ref[0],
                    preferred_element_type=jnp.float32) + b1_ref[0]
        h = 0.5 * h * (1.0 + lax.erf(h * (1.0 / math.sqrt(2.0))))
        out_ref[...] += jnp.dot(h, w2_ref[0],
                                preferred_element_type=jnp.float32)

    @pl.when(j == NF - 1)
    def _():
        out_ref[...] = (out_ref[...] + b2_ref[0]) * s_ref[...][:, 0:1]


@functools.cache
def _sc_kernels():
    mesh = plsc.VectorSubcoreMesh(core_axis_name="c", subcore_axis_name="s")

    sc_scratch = [
        pltpu.VMEM((NCHUNK, CH), jnp.int32),
        pltpu.VMEM((2, CH, D), jnp.float32),
        pltpu.SemaphoreType.DMA,
        pltpu.SemaphoreType.DMA,
        pltpu.SemaphoreType.DMA,
        pltpu.SemaphoreType.DMA,
    ]

    def _pipelined(start_in, start_out):
        lds = [None, None]
        sts = [None, None]
        lds[0] = start_in(0, 0)
        for c in range(NCHUNK):
            buf = c % 2
            if c + 1 < NCHUNK:
                nbuf = (c + 1) % 2
                if sts[nbuf] is not None:
                    sts[nbuf].wait()
                    sts[nbuf] = None
                lds[nbuf] = start_in(c + 1, nbuf)
            lds[buf].wait()
            sts[buf] = start_out(c, buf)
        for buf in range(2):
            if sts[buf] is not None:
                sts[buf].wait()

    @functools.partial(
        pl.kernel, mesh=mesh,
        out_type=[
            jax.ShapeDtypeStruct((TP, D), jnp.float32),
            jax.ShapeDtypeStruct((TP, 128), jnp.float32),
        ],
        scratch_types=sc_scratch + [
            pltpu.VMEM((NCHUNK, CH, 128), jnp.float32),
            pltpu.SemaphoreType.DMA,
        ],
    )
    def scatter_rows(x_hbm, pos_hbm, s_hbm, xs_hbm, ss_hbm, idx_v, rows,
                     li0, li1, lo0, lo1, sc_v, ssm):
        wid = lax.axis_index("s") * NC + lax.axis_index("c")
        base = wid * TPW
        pltpu.sync_copy(pos_hbm.at[wid], idx_v)
        pltpu.sync_copy(s_hbm.at[wid], sc_v)
        lsem = [li0, li1]
        ssem = [lo0, lo1]

        sc_copies = [
            pltpu.async_copy(sc_v.at[c], ss_hbm.at[idx_v.at[c]], ssm)
            for c in range(NCHUNK)
        ]

        def start_in(c, buf):
            return pltpu.async_copy(x_hbm.at[pl.ds(base + c * CH, CH)],
                                    rows.at[buf], lsem[buf])

        def start_out(c, buf):
            return pltpu.async_copy(rows.at[buf], xs_hbm.at[idx_v.at[c]],
                                    ssem[buf])

        _pipelined(start_in, start_out)
        for cp in sc_copies:
            cp.wait()

    @functools.partial(
        pl.kernel, mesh=mesh,
        out_type=jax.ShapeDtypeStruct((T, D), jnp.float32),
        scratch_types=sc_scratch,
    )
    def gather_rows(os_hbm, pos_hbm, out_hbm, idx_v, rows, li0, li1, lo0, lo1):
        wid = lax.axis_index("s") * NC + lax.axis_index("c")
        base = wid * TPW
        pltpu.sync_copy(pos_hbm.at[wid], idx_v)
        lsem = [li0, li1]
        ssem = [lo0, lo1]

        def start_in(c, buf):
            return pltpu.async_copy(os_hbm.at[idx_v.at[c]], rows.at[buf],
                                    lsem[buf])

        def start_out(c, buf):
            return pltpu.async_copy(rows.at[buf],
                                    out_hbm.at[pl.ds(base + c * CH, CH)],
                                    ssem[buf])

        _pipelined(start_in, start_out)

    return scatter_rows, gather_rows


def kernel(hidden_states, wg_reduction, wg, weight1, bias1, weight2, bias2):
    b, t, c = hidden_states.shape
    x = hidden_states.reshape(-1, c)

    scores, eidx, rank, counts = pl.pallas_call(
        _gate_kernel,
        grid=(T // TB,),
        in_specs=[
            pl.BlockSpec((TB, D), lambda i: (i, 0)),
            pl.BlockSpec((16, D), lambda i: (0, 0)),
            pl.BlockSpec((E, 16), lambda i: (0, 0)),
        ],
        out_specs=[
            pl.BlockSpec((TB, 128), lambda i: (i, 0)),
            pl.BlockSpec((TB, 1), lambda i: (i, 0)),
            pl.BlockSpec((TB, 1), lambda i: (i, 0)),
            pl.BlockSpec((1, E), lambda i: (0, 0)),
        ],
        out_shape=[
            jax.ShapeDtypeStruct((T, 128), jnp.float32),
            jax.ShapeDtypeStruct((T, 1), jnp.int32),
            jax.ShapeDtypeStruct((T, 1), jnp.int32),
            jax.ShapeDtypeStruct((1, E), jnp.float32),
        ],
    )(x, wg_reduction, wg)

    pos, be = pl.pallas_call(
        _route_kernel,
        out_shape=[
            jax.ShapeDtypeStruct((T, 1), jnp.int32),
            jax.ShapeDtypeStruct((NB, 1), jnp.int32),
        ],
    )(eidx, rank, counts)

    pos_w = pos.reshape(NW, NCHUNK, CH)
    be_flat = be.reshape(NB)
    scores_w = scores.reshape(NW, NCHUNK, CH, 128)

    scatter_rows, gather_rows = _sc_kernels()
    x_sorted, s_sorted = scatter_rows(x, pos_w, scores_w)

    def _jj(i, j):
        return jnp.where(i % 2 == 0, j, NF - 1 - j)

    grid_spec = pltpu.PrefetchScalarGridSpec(
        num_scalar_prefetch=1,
        grid=(NB, NF),
        in_specs=[
            pl.BlockSpec((TBLK, D), lambda i, j, be: (i, 0)),
            pl.BlockSpec((1, D, F),
                         lambda i, j, be: (jnp.minimum(be[i], E - 1), 0,
                                           _jj(i, j))),
            pl.BlockSpec((1, 1, F),
                         lambda i, j, be: (jnp.minimum(be[i], E - 1), 0,
                                           _jj(i, j))),
            pl.BlockSpec((1, F, D),
                         lambda i, j, be: (jnp.minimum(be[i], E - 1),
                                           _jj(i, j), 0)),
            pl.BlockSpec((1, 1, D),
                         lambda i, j, be: (jnp.minimum(be[i], E - 1), 0, 0)),
            pl.BlockSpec((TBLK, 128), lambda i, j, be: (i, 0)),
        ],
        out_specs=pl.BlockSpec((TBLK, D), lambda i, j, be: (i, 0)),
    )
    o_sorted = pl.pallas_call(
        _ffn_kernel,
        grid_spec=grid_spec,
        out_shape=jax.ShapeDtypeStruct((TP, D), jnp.float32),
    )(be_flat, x_sorted, weight1, bias1.reshape(E, 1, DFF), weight2,
      bias2.reshape(E, 1, D), s_sorted)

    out = gather_rows(o_sorted, pos_w)

    return out.reshape(b, t, c)


# speedup vs baseline: 1.1152x; 1.1152x over previous
import functools
import math

import jax
import jax.numpy as jnp
from jax import lax
from jax.experimental import pallas as pl
from jax.experimental.pallas import tpu as pltpu
from jax.experimental.pallas import tpu_sc as plsc

E = 8
D = 1024
DFF = 4096
T = 4096

TB = 512
TBLK = 512
NB = T // TBLK + E
TP = NB * TBLK
F = 2048
NF = DFF // F

NC = 2
NS = 16
NW = NC * NS
TPW = T // NW
CH = 32
NCHUNK = TPW // CH


def _gate_route_kernel(x_ref, wgr_ref, wg_ref, scores_ref, pos_ref, be_ref,
                       eidx_s, rank_s, counts_s, tri_s):
    i = pl.program_id(0)

    @pl.when(i == 0)
    def _():
        counts_s[...] = jnp.zeros_like(counts_s)
        tri_s[...] = (lax.broadcasted_iota(jnp.int32, (TB, TB), 0)
                      >= lax.broadcasted_iota(jnp.int32, (TB, TB), 1)
                      ).astype(jnp.float32)

    @pl.when(i < T // TB)
    def _():
        xb = x_ref[...]
        red = jnp.dot(xb, wgr_ref[...].T, preferred_element_type=jnp.float32)
        wg = wg_ref[...]
        wgu = wg * lax.rsqrt(jnp.sum(wg * wg, axis=1, keepdims=True))
        logits = jnp.dot(red, wgu.T, preferred_element_type=jnp.float32)
        m = jnp.max(logits, axis=1, keepdims=True)
        ssum = jnp.sum(jnp.exp(logits - m), axis=1, keepdims=True)
        scores_ref[...] = jnp.broadcast_to(1.0 / ssum, (TB, 128))
        it8 = lax.broadcasted_iota(jnp.int32, (TB, E), 1)
        eidx = jnp.min(jnp.where(logits == m, it8, E), axis=1, keepdims=True)
        eidx_s[pl.ds(i * TB, TB), :] = eidx
        oh = (it8 == eidx).astype(jnp.float32)
        cs = jnp.dot(tri_s[...], oh, preferred_element_type=jnp.float32)
        prev = counts_s[...]
        rank_s[pl.ds(i * TB, TB), :] = jnp.sum((cs + prev) * oh, axis=1,
                                               keepdims=True)
        counts_s[...] = prev + jnp.sum(oh, axis=0, keepdims=True)

    @pl.when(i == T // TB)
    def _():
        counts = counts_s[...]
        padded = jnp.floor((counts + (TBLK - 1)) / TBLK) * TBLK
        triu = (lax.broadcasted_iota(jnp.int32, (E, E), 0)
                <= lax.broadcasted_iota(jnp.int32, (E, E), 1)
                ).astype(jnp.float32)
        ends = jnp.dot(padded, triu, preferred_element_type=jnp.float32)
        starts = ends - padded
        eidx = eidx_s[...]
        oh = (lax.broadcasted_iota(jnp.int32, (T, E), 1)
              == eidx).astype(jnp.float32)
        start_t = jnp.sum(oh * starts, axis=1, keepdims=True)
        pos_ref[...] = (start_t + rank_s[...] - 1.0).astype(jnp.int32)
        ib = (lax.broadcasted_iota(jnp.int32, (NB, E), 0)
              * TBLK).astype(jnp.float32)
        be_ref[...] = jnp.sum((ib >= ends).astype(jnp.int32), axis=1,
                              keepdims=True)


def _ffn_kernel(be_ref, x_ref, w1_ref, b1_ref, w2_ref, b2_ref, s_ref, out_ref):
    i = pl.program_id(0)
    j = pl.program_id(1)

    @pl.when(j == 0)
    def _():
        out_ref[...] = jnp.zeros_like(out_ref)

    @pl.when(be_ref[i] < E)
    def _():
        h = jnp.dot(x_ref[...], w1_ref[0],
                    preferred_element_type=jnp.float32) + b1_ref[0]
        h = 0.5 * h * (1.0 + lax.erf(h * (1.0 / math.sqrt(2.0))))
        out_ref[...] += jnp.dot(h, w2_ref[0],
                                preferred_element_type=jnp.float32)

    @pl.when(j == NF - 1)
    def _():
        out_ref[...] = (out_ref[...] + b2_ref[0]) * s_ref[...][:, 0:1]


@functools.cache
def _sc_kernels():
    mesh = plsc.VectorSubcoreMesh(core_axis_name="c", subcore_axis_name="s")

    sc_scratch = [
        pltpu.VMEM((NCHUNK, CH), jnp.int32),
        pltpu.VMEM((2, CH, D), jnp.float32),
        pltpu.SemaphoreType.DMA,
        pltpu.SemaphoreType.DMA,
        pltpu.SemaphoreType.DMA,
        pltpu.SemaphoreType.DMA,
    ]

    def _pipelined(start_in, start_out):
        lds = [None, None]
        sts = [None, None]
        lds[0] = start_in(0, 0)
        for c in range(NCHUNK):
            buf = c % 2
            if c + 1 < NCHUNK:
                nbuf = (c + 1) % 2
                if sts[nbuf] is not None:
                    sts[nbuf].wait()
                    sts[nbuf] = None
                lds[nbuf] = start_in(c + 1, nbuf)
            lds[buf].wait()
            sts[buf] = start_out(c, buf)
        for buf in range(2):
            if sts[buf] is not None:
                sts[buf].wait()

    @functools.partial(
        pl.kernel, mesh=mesh,
        out_type=[
            jax.ShapeDtypeStruct((TP, D), jnp.float32),
            jax.ShapeDtypeStruct((TP, 128), jnp.float32),
        ],
        scratch_types=sc_scratch + [
            pltpu.VMEM((NCHUNK, CH, 128), jnp.float32),
            pltpu.SemaphoreType.DMA,
        ],
    )
    def scatter_rows(x_hbm, pos_hbm, s_hbm, xs_hbm, ss_hbm, idx_v, rows,
                     li0, li1, lo0, lo1, sc_v, ssm):
        wid = lax.axis_index("s") * NC + lax.axis_index("c")
        base = wid * TPW
        pltpu.sync_copy(pos_hbm.at[wid], idx_v)
        pltpu.sync_copy(s_hbm.at[wid], sc_v)
        lsem = [li0, li1]
        ssem = [lo0, lo1]

        sc_copies = [
            pltpu.async_copy(sc_v.at[c], ss_hbm.at[idx_v.at[c]], ssm)
            for c in range(NCHUNK)
        ]

        def start_in(c, buf):
            return pltpu.async_copy(x_hbm.at[pl.ds(base + c * CH, CH)],
                                    rows.at[buf], lsem[buf])

        def start_out(c, buf):
            return pltpu.async_copy(rows.at[buf], xs_hbm.at[idx_v.at[c]],
                                    ssem[buf])

        _pipelined(start_in, start_out)
        for cp in sc_copies:
            cp.wait()

    @functools.partial(
        pl.kernel, mesh=mesh,
        out_type=jax.ShapeDtypeStruct((T, D), jnp.float32),
        scratch_types=sc_scratch,
    )
    def gather_rows(os_hbm, pos_hbm, out_hbm, idx_v, rows, li0, li1, lo0, lo1):
        wid = lax.axis_index("s") * NC + lax.axis_index("c")
        base = wid * TPW
        pltpu.sync_copy(pos_hbm.at[wid], idx_v)
        lsem = [li0, li1]
        ssem = [lo0, lo1]

        def start_in(c, buf):
            return pltpu.async_copy(os_hbm.at[idx_v.at[c]], rows.at[buf],
                                    lsem[buf])

        def start_out(c, buf):
            return pltpu.async_copy(rows.at[buf],
                                    out_hbm.at[pl.ds(base + c * CH, CH)],
                                    ssem[buf])

        _pipelined(start_in, start_out)

    return scatter_rows, gather_rows


def kernel(hidden_states, wg_reduction, wg, weight1, bias1, weight2, bias2):
    b, t, c = hidden_states.shape
    x = hidden_states.reshape(-1, c)

    nsteps = T // TB
    scores, pos, be = pl.pallas_call(
        _gate_route_kernel,
        grid=(nsteps + 1,),
        in_specs=[
            pl.BlockSpec((TB, D), lambda i: (jnp.minimum(i, nsteps - 1), 0)),
            pl.BlockSpec((16, D), lambda i: (0, 0)),
            pl.BlockSpec((E, 16), lambda i: (0, 0)),
        ],
        out_specs=[
            pl.BlockSpec((TB, 128),
                         lambda i: (jnp.minimum(i, nsteps - 1), 0)),
            pl.BlockSpec((T, 1), lambda i: (0, 0)),
            pl.BlockSpec((NB, 1), lambda i: (0, 0)),
        ],
        out_shape=[
            jax.ShapeDtypeStruct((T, 128), jnp.float32),
            jax.ShapeDtypeStruct((T, 1), jnp.int32),
            jax.ShapeDtypeStruct((NB, 1), jnp.int32),
        ],
        scratch_shapes=[
            pltpu.VMEM((T, 1), jnp.int32),
            pltpu.VMEM((T, 1), jnp.float32),
            pltpu.VMEM((1, E), jnp.float32),
            pltpu.VMEM((TB, TB), jnp.float32),
        ],
    )(x, wg_reduction, wg)

    pos_w = pos.reshape(NW, NCHUNK, CH)
    be_flat = be.reshape(NB)
    scores_w = scores.reshape(NW, NCHUNK, CH, 128)

    scatter_rows, gather_rows = _sc_kernels()
    x_sorted, s_sorted = scatter_rows(x, pos_w, scores_w)

    def _jj(i, j):
        return jnp.where(i % 2 == 0, j, NF - 1 - j)

    grid_spec = pltpu.PrefetchScalarGridSpec(
        num_scalar_prefetch=1,
        grid=(NB, NF),
        in_specs=[
            pl.BlockSpec((TBLK, D), lambda i, j, be: (i, 0)),
            pl.BlockSpec((1, D, F),
                         lambda i, j, be: (jnp.minimum(be[i], E - 1), 0,
                                           _jj(i, j))),
            pl.BlockSpec((1, 1, F),
                         lambda i, j, be: (jnp.minimum(be[i], E - 1), 0,
                                           _jj(i, j))),
            pl.BlockSpec((1, F, D),
                         lambda i, j, be: (jnp.minimum(be[i], E - 1),
                                           _jj(i, j), 0)),
            pl.BlockSpec((1, 1, D),
                         lambda i, j, be: (jnp.minimum(be[i], E - 1), 0, 0)),
            pl.BlockSpec((TBLK, 128), lambda i, j, be: (i, 0)),
        ],
        out_specs=pl.BlockSpec((TBLK, D), lambda i, j, be: (i, 0)),
    )
    o_sorted = pl.pallas_call(
        _ffn_kernel,
        grid_spec=grid_spec,
        out_shape=jax.ShapeDtypeStruct((TP, D), jnp.float32),
    )(be_flat, x_sorted, weight1, bias1.reshape(E, 1, DFF), weight2,
      bias2.reshape(E, 1, D), s_sorted)

    out = gather_rows(o_sorted, pos_w)

    return out.reshape(b, t, c)
